# baseline (speedup 1.0000x reference)
import jax
import jax.numpy as jnp
from jax.experimental import pallas as pl
from jax.experimental.pallas import tpu as pltpu

_BN = 2304
_DIMS = (91, 12, 10, 8, 364)
_OFFS = (0, 96, 112, 128, 136)
_TOTAL = 500


def _heads_kernel(x_ref,
                  wc_ref, bc_ref, wco_ref, bco_ref, wm_ref, bm_ref,
                  ws_ref, bs_ref, wb_ref, bb_ref,
                  scores_ref, color_ref, material_ref, state_ref, bbox_ref,
                  w_s, b_s):
    w_refs = (wc_ref, wco_ref, wm_ref, ws_ref, wb_ref)
    b_refs = (bc_ref, bco_ref, bm_ref, bs_ref, bb_ref)

    @pl.when(pl.program_id(0) == 0)
    def _stack():
        for w_ref, b_ref, d, off in zip(w_refs, b_refs, _DIMS, _OFFS):
            w_s[off:off + d] = w_ref[...].astype(jnp.bfloat16)
            b_s[off:off + d] = b_ref[...][:, None]

    x = x_ref[...].astype(jnp.bfloat16)
    y = jax.lax.dot_general(w_s[...], x, (((1,), (1,)), ((), ())),
                            preferred_element_type=jnp.float32)
    y = y + b_s[...]
    for ref, d, off in zip(
            (scores_ref, color_ref, material_ref, state_ref, bbox_ref),
            _DIMS, _OFFS):
        ref[...] = y[off:off + d]


def kernel(x, W_cls, b_cls, W_color, b_color, W_material, b_material,
           W_state, b_state, W_bbox, b_bbox):
    n, c = x.shape
    heads = [(W_cls, b_cls), (W_color, b_color), (W_material, b_material),
             (W_state, b_state), (W_bbox, b_bbox)]

    grid = (pl.cdiv(n, _BN),)
    full = pl.BlockSpec(None, lambda i: (0, 0))
    vec = pl.BlockSpec(None, lambda i: (0,))
    in_specs = [pl.BlockSpec((_BN, c), lambda i: (i, 0))]
    operands = [x]
    for W, b in heads:
        in_specs += [full, vec]
        operands += [W.T, b]

    out_shapes = tuple(jax.ShapeDtypeStruct((d, n), jnp.float32)
                       for d in _DIMS)
    out_specs = tuple(pl.BlockSpec((d, _BN), lambda i: (0, i))
                      for d in _DIMS)

    outs = pl.pallas_call(
        _heads_kernel,
        grid=grid,
        in_specs=in_specs,
        out_specs=out_specs,
        out_shape=out_shapes,
        scratch_shapes=[pltpu.VMEM((_TOTAL, c), jnp.bfloat16),
                        pltpu.VMEM((_TOTAL, 1), jnp.float32)],
        compiler_params=pltpu.CompilerParams(
            dimension_semantics=("arbitrary",)),
    )(*operands)
    return tuple(jnp.transpose(o) for o in outs)


# speedup vs baseline: 1.0329x; 1.0329x over previous
import jax
import jax.numpy as jnp
from jax.experimental import pallas as pl
from jax.experimental.pallas import tpu as pltpu

_BN = 2560
_DIMS = (91, 12, 10, 8, 364)
_OFFS = (0, 96, 112, 128, 136)
_TOTAL = 500


def _heads_kernel(x_ref,
                  wc_ref, bc_ref, wco_ref, bco_ref, wm_ref, bm_ref,
                  ws_ref, bs_ref, wb_ref, bb_ref,
                  scores_ref, color_ref, material_ref, state_ref, bbox_ref,
                  w_s, b_s):
    w_refs = (wc_ref, wco_ref, wm_ref, ws_ref, wb_ref)
    b_refs = (bc_ref, bco_ref, bm_ref, bs_ref, bb_ref)

    @pl.when(pl.program_id(0) == 0)
    def _stack():
        for w_ref, b_ref, d, off in zip(w_refs, b_refs, _DIMS, _OFFS):
            w_s[off:off + d] = w_ref[...].astype(jnp.bfloat16)
            b_s[off:off + d] = b_ref[...][:, None]

    x = x_ref[...].astype(jnp.bfloat16)
    y = jax.lax.dot_general(w_s[...], x, (((1,), (1,)), ((), ())),
                            preferred_element_type=jnp.float32)
    y = y + b_s[...]
    for ref, d, off in zip(
            (scores_ref, color_ref, material_ref, state_ref, bbox_ref),
            _DIMS, _OFFS):
        ref[...] = y[off:off + d]


def kernel(x, W_cls, b_cls, W_color, b_color, W_material, b_material,
           W_state, b_state, W_bbox, b_bbox):
    n, c = x.shape
    heads = [(W_cls, b_cls), (W_color, b_color), (W_material, b_material),
             (W_state, b_state), (W_bbox, b_bbox)]

    grid = (pl.cdiv(n, _BN),)
    full = pl.BlockSpec(None, lambda i: (0, 0))
    vec = pl.BlockSpec(None, lambda i: (0,))
    in_specs = [pl.BlockSpec((_BN, c), lambda i: (i, 0))]
    operands = [x]
    for W, b in heads:
        in_specs += [full, vec]
        operands += [W.T, b]

    out_shapes = tuple(jax.ShapeDtypeStruct((d, n), jnp.float32)
                       for d in _DIMS)
    out_specs = tuple(pl.BlockSpec((d, _BN), lambda i: (0, i))
                      for d in _DIMS)

    outs = pl.pallas_call(
        _heads_kernel,
        grid=grid,
        in_specs=in_specs,
        out_specs=out_specs,
        out_shape=out_shapes,
        scratch_shapes=[pltpu.VMEM((_TOTAL, c), jnp.bfloat16),
                        pltpu.VMEM((_TOTAL, 1), jnp.float32)],
        compiler_params=pltpu.CompilerParams(
            dimension_semantics=("arbitrary",)),
    )(*operands)
    return tuple(jnp.transpose(o) for o in outs)
